# baseline (speedup 1.0000x reference)
import functools

import jax
import jax.numpy as jnp
from jax import lax
from jax.experimental import pallas as pl
from jax.experimental.pallas import tpu as pltpu
from jax.experimental.pallas import tpu_sc as plsc

ROWS = 1024
COLS = 100000
KTOP = 5

LANES = 16
NCORES = 2
NSUB = 16
NWORKERS = NCORES * NSUB
RPW = ROWS // NWORKERS
GW = 128
CMAX = COLS - 160

CBLK = 4096
KCH = CBLK // 128
NBLK = -(-COLS // CBLK)
LASTB = NBLK - 1
TAILV = COLS - LASTB * CBLK
TAILK = TAILV // 128
TAILR = TAILV - TAILK * 128
TSTART = LASTB * CBLK
TCUT = CMAX + GW


RPS = ROWS // NCORES


def _sc_gather_body(logits2d, targets_hbm, chunks_out, tgt_s, sem):
    cid = lax.axis_index("c")
    base = cid * RPS
    pltpu.sync_copy(targets_hbm.at[pl.ds(base, RPS)], tgt_s)

    def issue(i, _):
        t_s = tgt_s[i]
        col0 = pl.multiple_of(
            jnp.minimum(lax.bitwise_and(t_s, -GW), CMAX), GW)
        row = base + i
        r0 = pl.multiple_of(lax.bitwise_and(row, -8), 8)
        pltpu.async_copy(
            logits2d.at[pl.ds(r0, 8), pl.ds(col0, GW)],
            chunks_out.at[row], sem)
        return 0

    lax.fori_loop(0, RPS, issue, 0, unroll=8)

    def drain(i, _):
        pltpu.make_async_copy(
            logits2d.at[pl.ds(0, 8), pl.ds(0, GW)],
            chunks_out.at[0], sem).wait()
        return 0

    lax.fori_loop(0, RPS, drain, 0, unroll=8)


@functools.lru_cache(maxsize=1)
def _sc_gather():
    mesh = plsc.ScalarSubcoreMesh(axis_name="c", num_cores=NCORES)
    return pl.kernel(
        _sc_gather_body,
        out_type=jax.ShapeDtypeStruct((ROWS, 8, GW), jnp.float32),
        mesh=mesh,
        scratch_types=[
            pltpu.SMEM((RPS,), jnp.int32),
            pltpu.SemaphoreType.DMA,
        ],
    )


def _tc_body(logits_ref, chunks_ref, tgt_ref, pm_ref, out_ref, acc_ref,
             thr_ref, thi_ref, tgtb_ref):
    c = pl.program_id(0)
    b = jnp.where(c == 0, LASTB, c - 1)

    colv = lax.broadcasted_iota(jnp.int32, (ROWS, 128), 1)

    def slice_count(k, extra_mask=None):
        vk = logits_ref[:, k * 128:(k + 1) * 128]
        mlt = colv < (tgtb_ref[...] - (b * CBLK + k * 128))
        m = vk >= jnp.where(mlt, thr_ref[...], thi_ref[...])
        if extra_mask is not None:
            m = m & extra_mask
        return m.astype(jnp.int32)

    @pl.when(c == 0)
    def _init():
        tgt = tgt_ref[...]
        col0 = jnp.minimum(tgt & -GW, CMAX)
        lane3 = (tgt - col0).reshape(ROWS, 1, 1)
        r3 = lax.broadcasted_iota(jnp.int32, (ROWS, 8, GW), 0)
        s3 = lax.broadcasted_iota(jnp.int32, (ROWS, 8, GW), 1)
        l3 = lax.broadcasted_iota(jnp.int32, (ROWS, 8, GW), 2)
        oh3 = (s3 == (r3 & 7)) & (l3 == lane3)
        t_chunk = jnp.sum(
            jnp.sum(jnp.where(oh3, chunks_ref[...], 0.0), axis=2),
            axis=1, keepdims=True)
        tgt_rel = tgt - TSTART
        vk = logits_ref[:, TAILK * 128:(TAILK + 1) * 128]
        oh = colv == (tgt_rel - TAILK * 128)
        hit = jnp.sum(jnp.where(oh, vk, 0.0), axis=1, keepdims=True)
        t = jnp.where(tgt >= TCUT, hit, t_chunk)
        thr_ref[...] = jnp.broadcast_to(t, (ROWS, 128))
        bb = lax.bitcast_convert_type(t + 0.0, jnp.int32)
        bhi = jnp.where(bb >= 0, bb + 1, bb - 1)
        thi_ref[...] = jnp.broadcast_to(
            lax.bitcast_convert_type(bhi, jnp.float32), (ROWS, 128))
        tgtb_ref[...] = jnp.broadcast_to(tgt, (ROWS, 128))
        s = slice_count(0)
        for k in range(1, TAILK):
            s += slice_count(k)
        s += slice_count(TAILK, extra_mask=colv < TAILR)
        acc_ref[...] = s

    @pl.when(c > 0)
    def _main():
        s = slice_count(0)
        for k in range(1, KCH):
            s += slice_count(k)
        acc_ref[...] += s

    @pl.when(c == NBLK - 1)
    def _fin():
        cnt = jnp.sum(acc_ref[...], axis=1, keepdims=True)
        pm = pm_ref[...]
        correct = jnp.where(cnt < KTOP, pm, 0.0)
        out_ref[0, 0] = jnp.sum(correct) / jnp.sum(pm)


def _tc_accuracy(logits, chunks, tgt2, pm2):
    return pl.pallas_call(
        _tc_body,
        grid=(NBLK,),
        in_specs=[
            pl.BlockSpec((ROWS, CBLK),
                         lambda c: (0, jnp.where(c == 0, LASTB, c - 1))),
            pl.BlockSpec((ROWS, 8, GW), lambda c: (0, 0, 0)),
            pl.BlockSpec((ROWS, 1), lambda c: (0, 0)),
            pl.BlockSpec((ROWS, 1), lambda c: (0, 0)),
        ],
        out_specs=pl.BlockSpec(memory_space=pltpu.SMEM),
        out_shape=jax.ShapeDtypeStruct((1, 1), jnp.float32),
        scratch_shapes=[
            pltpu.VMEM((ROWS, 128), jnp.int32),
            pltpu.VMEM((ROWS, 128), jnp.float32),
            pltpu.VMEM((ROWS, 128), jnp.float32),
            pltpu.VMEM((ROWS, 128), jnp.int32),
        ],
    )(logits, chunks, tgt2, pm2)


def kernel(logits, targets, padding_mask):
    tgt = targets.astype(jnp.int32)
    chunks = _sc_gather()(logits, tgt)
    acc = _tc_accuracy(
        logits,
        chunks,
        tgt.reshape(ROWS, 1),
        padding_mask.astype(jnp.float32).reshape(ROWS, 1),
    )
    return acc[0, 0]
